# Initial kernel scaffold; baseline (speedup 1.0000x reference)
import jax
import jax.numpy as jnp
from jax.experimental import pallas as pl


def kernel(x, adj_t):
    raise NotImplementedError("write your pallas kernel here")


# speedup vs baseline: 17.5674x; 17.5674x over previous
import functools

import jax
import jax.numpy as jnp
from jax import lax
from jax.experimental import pallas as pl
from jax.experimental.pallas import tpu as pltpu
from jax.experimental.pallas import tpu_sc as plsc

N_NODES = 10000
D = 128
N_EDGES = 320000

NC = 2
NS = 16
NW = NC * NS

NPAD = 10240
ROWS_PER_TILE = NPAD // NS

CHUNK = 128
E2 = 2 * N_EDGES
STEPS = -(-E2 // (NW * CHUNK))
E_PAD = NW * STEPS * CHUNK
EDGES_PER_TILE = STEPS * CHUNK

_mesh = plsc.VectorSubcoreMesh(core_axis_name="c", subcore_axis_name="s")


@functools.partial(
    pl.kernel,
    out_type=jax.ShapeDtypeStruct((NC, NPAD), jnp.float32),
    mesh=_mesh,
    scratch_types=[
        pltpu.VMEM_SHARED((NPAD,), jnp.float32),
        pltpu.VMEM((CHUNK,), jnp.int32),
        pltpu.VMEM((CHUNK,), jnp.float32),
    ],
)
def _sc_degree(col_hbm, zeros1_hbm, deg_out, deg_sh, cidx_v, ones_v):
    c = lax.axis_index("c")
    s = lax.axis_index("s")
    wid = c * NS + s
    for j in range(CHUNK // 16):
        ones_v[pl.ds(16 * j, 16)] = jnp.ones((16,), jnp.float32)
    pltpu.sync_copy(zeros1_hbm.at[pl.ds(s * ROWS_PER_TILE, ROWS_PER_TILE)],
                    deg_sh.at[pl.ds(s * ROWS_PER_TILE, ROWS_PER_TILE)])
    plsc.subcore_barrier()

    def step(i, carry):
        b = wid * EDGES_PER_TILE + i * CHUNK
        pltpu.sync_copy(col_hbm.at[pl.ds(b, CHUNK)], cidx_v)
        pltpu.sync_copy(ones_v, deg_sh.at[cidx_v], add=True)
        return carry

    lax.fori_loop(0, STEPS, step, 0)
    plsc.subcore_barrier()
    pltpu.sync_copy(deg_sh.at[pl.ds(s * ROWS_PER_TILE, ROWS_PER_TILE)],
                    deg_out.at[c, pl.ds(s * ROWS_PER_TILE, ROWS_PER_TILE)])


@functools.partial(
    pl.kernel,
    out_type=jax.ShapeDtypeStruct((NC, NPAD, D), jnp.float32),
    mesh=_mesh,
    scratch_types=[
        pltpu.VMEM_SHARED((NPAD, D), jnp.float32),
        pltpu.VMEM((CHUNK,), jnp.int32),
        pltpu.VMEM((CHUNK,), jnp.int32),
        pltpu.VMEM((CHUNK, D), jnp.float32),
        pltpu.SemaphoreType.DMA,
    ],
)
def _sc_propagate(y_hbm, row_hbm, col_hbm, zeros2_hbm, z_out,
                  acc_sh, ridx_v, cidx_v, rows_v, sem):
    c = lax.axis_index("c")
    s = lax.axis_index("s")
    wid = c * NS + s
    pltpu.sync_copy(zeros2_hbm.at[pl.ds(s * ROWS_PER_TILE, ROWS_PER_TILE)],
                    acc_sh.at[pl.ds(s * ROWS_PER_TILE, ROWS_PER_TILE)])
    plsc.subcore_barrier()

    def step(i, carry):
        b = wid * EDGES_PER_TILE + i * CHUNK
        pltpu.sync_copy(row_hbm.at[pl.ds(b, CHUNK)], ridx_v)
        pltpu.sync_copy(col_hbm.at[pl.ds(b, CHUNK)], cidx_v)
        pltpu.async_copy(y_hbm.at[ridx_v], rows_v, sem).wait()
        pltpu.sync_copy(rows_v, acc_sh.at[cidx_v], add=True)
        return carry

    lax.fori_loop(0, STEPS, step, 0)
    plsc.subcore_barrier()
    pltpu.sync_copy(acc_sh.at[pl.ds(s * ROWS_PER_TILE, ROWS_PER_TILE)],
                    z_out.at[c, pl.ds(s * ROWS_PER_TILE, ROWS_PER_TILE)])


_GRID = 8
_RB = NPAD // _GRID


def _tc_norm_body(dp_ref, x_ref, dis_ref, y0_ref):
    deg = dp_ref[0] + dp_ref[1]
    dis = jnp.where(deg > 0.0, lax.rsqrt(deg), 0.0)
    dis_ref[...] = dis
    y0_ref[...] = x_ref[...] * dis


_tc_norm = pl.pallas_call(
    _tc_norm_body,
    grid=(_GRID,),
    in_specs=[
        pl.BlockSpec((2, _RB, 1), lambda i: (0, i, 0)),
        pl.BlockSpec((_RB, D), lambda i: (i, 0)),
    ],
    out_specs=[
        pl.BlockSpec((_RB, 1), lambda i: (i, 0)),
        pl.BlockSpec((_RB, D), lambda i: (i, 0)),
    ],
    out_shape=[
        jax.ShapeDtypeStruct((NPAD, 1), jnp.float32),
        jax.ShapeDtypeStruct((NPAD, D), jnp.float32),
    ],
)


def _tc_mid_body(zp_ref, dis_ref, x_ref, y1_ref, acc_ref):
    d = dis_ref[...]
    x1 = (zp_ref[0] + zp_ref[1]) * d
    y1_ref[...] = x1 * d
    acc_ref[...] = x_ref[...] + x1


_tc_mid = pl.pallas_call(
    _tc_mid_body,
    grid=(_GRID,),
    in_specs=[
        pl.BlockSpec((2, _RB, D), lambda i: (0, i, 0)),
        pl.BlockSpec((_RB, 1), lambda i: (i, 0)),
        pl.BlockSpec((_RB, D), lambda i: (i, 0)),
    ],
    out_specs=[
        pl.BlockSpec((_RB, D), lambda i: (i, 0)),
        pl.BlockSpec((_RB, D), lambda i: (i, 0)),
    ],
    out_shape=[
        jax.ShapeDtypeStruct((NPAD, D), jnp.float32),
        jax.ShapeDtypeStruct((NPAD, D), jnp.float32),
    ],
)


def _tc_final_body(zp_ref, dis_ref, acc_ref, out_ref):
    x2 = (zp_ref[0] + zp_ref[1]) * dis_ref[...]
    out_ref[...] = (acc_ref[...] + x2) * jnp.float32(1.0 / 3.0)


_tc_final = pl.pallas_call(
    _tc_final_body,
    grid=(_GRID,),
    in_specs=[
        pl.BlockSpec((2, _RB, D), lambda i: (0, i, 0)),
        pl.BlockSpec((_RB, 1), lambda i: (i, 0)),
        pl.BlockSpec((_RB, D), lambda i: (i, 0)),
    ],
    out_specs=pl.BlockSpec((_RB, D), lambda i: (i, 0)),
    out_shape=jax.ShapeDtypeStruct((NPAD, D), jnp.float32),
)


def kernel(x, adj_t):
    adj = adj_t.astype(jnp.int32)
    pad_idx = jnp.full((E_PAD - E2,), N_NODES, jnp.int32)
    row = jnp.concatenate([adj[0], adj[1], pad_idx])
    col = jnp.concatenate([adj[1], adj[0], pad_idx])
    x_p = jnp.pad(x, ((0, NPAD - N_NODES), (0, 0)))
    zeros1 = jnp.zeros((NPAD,), jnp.float32)
    zeros2 = jnp.zeros((NPAD, D), jnp.float32)

    deg_parts = _sc_degree(col, zeros1)
    dis, y0 = _tc_norm(deg_parts.reshape(NC, NPAD, 1), x_p)

    z1 = _sc_propagate(y0, row, col, zeros2)
    y1, acc = _tc_mid(z1, dis, x_p)

    z2 = _sc_propagate(y1, row, col, zeros2)
    out = _tc_final(z2, dis, acc)
    return out[:N_NODES]
